# baseline (speedup 1.0000x reference)
import functools

import jax
import jax.numpy as jnp
from jax import lax
from jax.experimental import pallas as pl
from jax.experimental.pallas import tpu as pltpu
from jax.experimental.pallas import tpu_sc as plsc

BATCH = 16384
EMB = 32
HID = 10
PACK = 4
NC = 2
NS = 16
NW = NC * NS
BPW = BATCH // NW


def _gather_body(tu_hbm, tv_hbm, ue2_hbm, ve2_hbm, uout_hbm, vout_hbm,
                 tidx, rows, sem):
    wid = lax.axis_index("s") * NC + lax.axis_index("c")
    base = wid * BPW
    for (t_hbm, tbl, out_hbm) in ((tu_hbm, ue2_hbm, uout_hbm),
                                  (tv_hbm, ve2_hbm, vout_hbm)):
        pltpu.sync_copy(t_hbm.at[pl.ds(base, BPW)], tidx)
        pltpu.async_copy(tbl.at[tidx], rows, sem).wait()
        pltpu.sync_copy(rows, out_hbm.at[pl.ds(base, BPW)])


@jax.jit
def _sc_gather(tid_u, tid_v, ue2, ve2):
    mesh = plsc.VectorSubcoreMesh(core_axis_name="c", subcore_axis_name="s")
    f = pl.kernel(
        _gather_body,
        out_type=[
            jax.ShapeDtypeStruct((BATCH, 128), jnp.float32),
            jax.ShapeDtypeStruct((BATCH, 128), jnp.float32),
        ],
        mesh=mesh,
        scratch_types=[
            pltpu.VMEM((BPW,), jnp.int32),
            pltpu.VMEM((BPW, 128), jnp.float32),
            pltpu.SemaphoreType.DMA,
        ],
    )
    return f(tid_u, tid_v, ue2, ve2)


def _mlp_body(au_ref, av_ref, ru_ref, rv_ref, w1u_ref, w1v_ref,
              b1_ref, w2_ref, b2_ref, o_ref):
    def select(a_col, rows_ref):
        x = jnp.zeros((rows_ref.shape[0], EMB), jnp.float32)
        for a in range(PACK):
            m = (a_col == a).astype(jnp.float32)
            x += m * rows_ref[:, a * EMB:(a + 1) * EMB]
        return jnp.maximum(x, 0.0)

    U = select(au_ref[...], ru_ref)
    V = select(av_ref[...], rv_ref)
    h = jnp.dot(U, w1u_ref[...], preferred_element_type=jnp.float32)
    h += jnp.dot(V, w1v_ref[...], preferred_element_type=jnp.float32)
    h = jnp.maximum(h + b1_ref[...], 0.0)
    o_ref[...] = jnp.sum(h * w2_ref[...], axis=1, keepdims=True) + b2_ref[...]


@functools.partial(jax.jit, static_argnames=("bb",))
def _tc_mlp(au, av, ru, rv, w1u, w1v, b1, w2, b2, bb=4096):
    grid = (BATCH // bb,)
    return pl.pallas_call(
        _mlp_body,
        grid=grid,
        in_specs=[
            pl.BlockSpec((bb, 1), lambda i: (i, 0)),
            pl.BlockSpec((bb, 1), lambda i: (i, 0)),
            pl.BlockSpec((bb, 128), lambda i: (i, 0)),
            pl.BlockSpec((bb, 128), lambda i: (i, 0)),
            pl.BlockSpec((EMB, HID), lambda i: (0, 0)),
            pl.BlockSpec((EMB, HID), lambda i: (0, 0)),
            pl.BlockSpec((1, HID), lambda i: (0, 0)),
            pl.BlockSpec((1, HID), lambda i: (0, 0)),
            pl.BlockSpec((1, 1), lambda i: (0, 0)),
        ],
        out_specs=pl.BlockSpec((bb, 1), lambda i: (i, 0)),
        out_shape=jax.ShapeDtypeStruct((BATCH, 1), jnp.float32),
    )(au, av, ru, rv, w1u, w1v, b1, w2, b2)


def kernel(u, v, user_emb, item_emb, W1, b1, W2, b2):
    u = u.astype(jnp.int32)
    v = v.astype(jnp.int32)
    ue2 = user_emb.reshape(user_emb.shape[0] // PACK, EMB * PACK)
    ve2 = item_emb.reshape(item_emb.shape[0] // PACK, EMB * PACK)
    ru, rv = _sc_gather(u >> 2, v >> 2, ue2, ve2)
    return _tc_mlp((u & 3).reshape(BATCH, 1), (v & 3).reshape(BATCH, 1),
                   ru, rv, W1[:, :EMB].T, W1[:, EMB:].T,
                   b1.reshape(1, HID), W2.reshape(1, HID), b2.reshape(1, 1))


# speedup vs baseline: 1.0013x; 1.0013x over previous
import functools

import jax
import jax.numpy as jnp
from jax import lax
from jax.experimental import pallas as pl
from jax.experimental.pallas import tpu as pltpu
from jax.experimental.pallas import tpu_sc as plsc

BATCH = 16384
EMB = 32
HID = 10
NC = 2
NS = 16
NW = NC * NS
BPW = BATCH // NW
CH = 128
NCHUNK = BPW // CH
L = 16


def _extract_chunk(r8, gbuf, out_ref, k):
    lanes = lax.iota(jnp.int32, L)
    for g in range(CH // L):
        r = r8[pl.ds(k * CH + g * L, L)]
        s = lanes + g * L
        for c in range(EMB):
            col = jnp.full((L,), c, jnp.int32)
            val = plsc.load_gather(gbuf, [s, r, col])
            out_ref[c, pl.ds(k * CH + g * L, L)] = jnp.maximum(val, 0.0)


def _gather_body(tu_hbm, tv_hbm, ru_hbm, rv_hbm, ue3_hbm, ve3_hbm,
                 ut_hbm, vt_hbm,
                 tidx, r8, gb0, gb1, out_t, sem0, sem1):
    wid = lax.axis_index("s") * NC + lax.axis_index("c")
    base = wid * BPW

    for (t_hbm, rr_hbm, tbl, out_hbm) in (
            (tu_hbm, ru_hbm, ue3_hbm, ut_hbm),
            (tv_hbm, rv_hbm, ve3_hbm, vt_hbm)):
        pltpu.sync_copy(t_hbm.at[pl.ds(base, BPW)], tidx)
        pltpu.sync_copy(rr_hbm.at[pl.ds(base, BPW)], r8)
        pltpu.async_copy(tbl.at[tidx.at[pl.ds(0, CH)]], gb0, sem0)
        pltpu.async_copy(tbl.at[tidx.at[pl.ds(CH, CH)]], gb1, sem1)

        def chunk_pair(jj, carry, tbl=tbl):
            j0 = 2 * jj
            pltpu.make_async_copy(tbl.at[tidx.at[pl.ds(0, CH)]], gb0, sem0).wait()
            _extract_chunk(r8, gb0, out_t, j0)

            @pl.when(j0 + 2 < NCHUNK)
            def _():
                pltpu.async_copy(tbl.at[tidx.at[pl.ds((j0 + 2) * CH, CH)]],
                                 gb0, sem0)

            pltpu.make_async_copy(tbl.at[tidx.at[pl.ds(0, CH)]], gb1, sem1).wait()
            _extract_chunk(r8, gb1, out_t, j0 + 1)

            @pl.when(j0 + 3 < NCHUNK)
            def _():
                pltpu.async_copy(tbl.at[tidx.at[pl.ds((j0 + 3) * CH, CH)]],
                                 gb1, sem1)
            return carry

        lax.fori_loop(0, NCHUNK // 2, chunk_pair, None)
        pltpu.sync_copy(out_t, out_hbm.at[:, pl.ds(base, BPW)])


@jax.jit
def _sc_gather(tid_u, tid_v, r8_u, r8_v, ue3, ve3):
    mesh = plsc.VectorSubcoreMesh(core_axis_name="c", subcore_axis_name="s")
    f = pl.kernel(
        _gather_body,
        out_type=[
            jax.ShapeDtypeStruct((EMB, BATCH), jnp.float32),
            jax.ShapeDtypeStruct((EMB, BATCH), jnp.float32),
        ],
        mesh=mesh,
        scratch_types=[
            pltpu.VMEM((BPW,), jnp.int32),
            pltpu.VMEM((BPW,), jnp.int32),
            pltpu.VMEM((CH, 8, EMB), jnp.float32),
            pltpu.VMEM((CH, 8, EMB), jnp.float32),
            pltpu.VMEM((EMB, BPW), jnp.float32),
            pltpu.SemaphoreType.DMA,
            pltpu.SemaphoreType.DMA,
        ],
        compiler_params=pltpu.CompilerParams(
            needs_layout_passes=False, use_tc_tiling_on_sc=False),
    )
    return f(tid_u, tid_v, r8_u, r8_v, ue3, ve3)


def _mlp_body(ut_ref, vt_ref, w1u_ref, w1v_ref, b1_ref, w2_ref, b2_ref, o_ref):
    h = jnp.dot(w1u_ref[...], ut_ref[...], preferred_element_type=jnp.float32)
    h += jnp.dot(w1v_ref[...], vt_ref[...], preferred_element_type=jnp.float32)
    h = jnp.maximum(h + b1_ref[...], 0.0)
    o_ref[...] = jnp.dot(w2_ref[...], h, preferred_element_type=jnp.float32) + b2_ref[...]


@functools.partial(jax.jit, static_argnames=("bb",))
def _tc_mlp(ut, vt, w1u, w1v, b1, w2, b2, bb=4096):
    grid = (BATCH // bb,)
    return pl.pallas_call(
        _mlp_body,
        grid=grid,
        in_specs=[
            pl.BlockSpec((EMB, bb), lambda i: (0, i)),
            pl.BlockSpec((EMB, bb), lambda i: (0, i)),
            pl.BlockSpec((HID, EMB), lambda i: (0, 0)),
            pl.BlockSpec((HID, EMB), lambda i: (0, 0)),
            pl.BlockSpec((HID, 1), lambda i: (0, 0)),
            pl.BlockSpec((1, HID), lambda i: (0, 0)),
            pl.BlockSpec((1, 1), lambda i: (0, 0)),
        ],
        out_specs=pl.BlockSpec((1, bb), lambda i: (0, i)),
        out_shape=jax.ShapeDtypeStruct((1, BATCH), jnp.float32),
    )(ut, vt, w1u, w1v, b1, w2, b2)


def kernel(u, v, user_emb, item_emb, W1, b1, W2, b2):
    u = u.astype(jnp.int32)
    v = v.astype(jnp.int32)
    ue3 = user_emb.reshape(user_emb.shape[0] // 8, 8, EMB)
    ve3 = item_emb.reshape(item_emb.shape[0] // 8, 8, EMB)
    ut, vt = _sc_gather(u >> 3, v >> 3, u & 7, v & 7, ue3, ve3)
    out_t = _tc_mlp(ut, vt, W1[:, :EMB], W1[:, EMB:],
                    b1.reshape(HID, 1), W2, b2.reshape(1, 1))
    return out_t.reshape(BATCH, 1)


# speedup vs baseline: 2.3459x; 2.3429x over previous
import functools

import jax
import jax.numpy as jnp
from jax import lax
from jax.experimental import pallas as pl
from jax.experimental.pallas import tpu as pltpu
from jax.experimental.pallas import tpu_sc as plsc

BATCH = 16384
EMB = 32
HID = 10
NC = 2
NS = 16
NW = NC * NS
BPW = BATCH // NW
CH = 32
NCHUNK = BPW // CH
L = 16


def _fire_chunk(tbl, raw, gbuf, sem, base, k):
    for g in range(CH // L):
        vec = raw[pl.ds(k * CH + g * L, L)]
        for i in range(L):
            t = lax.shift_right_logical(vec[i], 3)
            pltpu.async_copy(tbl.at[pl.ds(t, 1)],
                             gbuf.at[pl.ds(g * L + i, 1)], sem)


def _extract_chunk(raw, gbuf, out_ref, k):
    lanes = lax.iota(jnp.int32, L)
    for g in range(CH // L):
        r = raw[pl.ds(k * CH + g * L, L)] & 7
        s = lanes + g * L
        for c in range(EMB):
            col = jnp.full((L,), c, jnp.int32)
            val = plsc.load_gather(gbuf, [s, r, col])
            out_ref[c, pl.ds(k * CH + g * L, L)] = jnp.maximum(val, 0.0)


def _gather_body(u_hbm, v_hbm, ue_hbm, ve_hbm, ut_hbm, vt_hbm,
                 raw_u, raw_v, gb0, gb1, out_u, out_v,
                 sem0, sem1, semo_u, semo_v):
    wid = lax.axis_index("s") * NC + lax.axis_index("c")
    base = wid * BPW
    pltpu.sync_copy(u_hbm.at[pl.ds(base, BPW)], raw_u)
    pltpu.sync_copy(v_hbm.at[pl.ds(base, BPW)], raw_v)

    for (tbl, raw, out_vmem, out_hbm, semo) in (
            (ue_hbm, raw_u, out_u, ut_hbm, semo_u),
            (ve_hbm, raw_v, out_v, vt_hbm, semo_v)):
        _fire_chunk(tbl, raw, gb0, sem0, base, 0)

        def chunk_pair(jj, carry, tbl=tbl, raw=raw, out_vmem=out_vmem):
            j0 = 2 * jj
            _fire_chunk(tbl, raw, gb1, sem1, base, j0 + 1)
            pltpu.make_async_copy(tbl.at[pl.ds(0, CH)], gb0, sem0).wait()
            _extract_chunk(raw, gb0, out_vmem, j0)

            @pl.when(j0 + 2 < NCHUNK)
            def _():
                _fire_chunk(tbl, raw, gb0, sem0, base, j0 + 2)

            pltpu.make_async_copy(tbl.at[pl.ds(0, CH)], gb1, sem1).wait()
            _extract_chunk(raw, gb1, out_vmem, j0 + 1)
            return carry

        lax.fori_loop(0, NCHUNK // 2, chunk_pair, None)
        pltpu.async_copy(out_vmem, out_hbm.at[:, pl.ds(base, BPW)], semo)

    pltpu.make_async_copy(out_u, ut_hbm.at[:, pl.ds(base, BPW)], semo_u).wait()
    pltpu.make_async_copy(out_v, vt_hbm.at[:, pl.ds(base, BPW)], semo_v).wait()


@jax.jit
def _sc_gather(u, v, ue3, ve3):
    mesh = plsc.VectorSubcoreMesh(core_axis_name="c", subcore_axis_name="s")
    f = pl.kernel(
        _gather_body,
        out_type=[
            jax.ShapeDtypeStruct((EMB, BATCH), jnp.float32),
            jax.ShapeDtypeStruct((EMB, BATCH), jnp.float32),
        ],
        mesh=mesh,
        scratch_types=[
            pltpu.VMEM((BPW,), jnp.int32),
            pltpu.VMEM((BPW,), jnp.int32),
            pltpu.VMEM((CH, 8, EMB), jnp.float32),
            pltpu.VMEM((CH, 8, EMB), jnp.float32),
            pltpu.VMEM((EMB, BPW), jnp.float32),
            pltpu.VMEM((EMB, BPW), jnp.float32),
            pltpu.SemaphoreType.DMA,
            pltpu.SemaphoreType.DMA,
            pltpu.SemaphoreType.DMA,
            pltpu.SemaphoreType.DMA,
        ],
        compiler_params=pltpu.CompilerParams(needs_layout_passes=False),
    )
    return f(u, v, ue3, ve3)


def _mlp_body(ut_ref, vt_ref, w1u_ref, w1v_ref, b1_ref, w2_ref, b2_ref, o_ref):
    h = jnp.dot(w1u_ref[...], ut_ref[...], preferred_element_type=jnp.float32)
    h += jnp.dot(w1v_ref[...], vt_ref[...], preferred_element_type=jnp.float32)
    h = jnp.maximum(h + b1_ref[...], 0.0)
    o_ref[...] = jnp.dot(w2_ref[...], h, preferred_element_type=jnp.float32) + b2_ref[...]


@functools.partial(jax.jit, static_argnames=("bb",))
def _tc_mlp(ut, vt, w1u, w1v, b1, w2, b2, bb=4096):
    grid = (BATCH // bb,)
    return pl.pallas_call(
        _mlp_body,
        grid=grid,
        in_specs=[
            pl.BlockSpec((EMB, bb), lambda i: (0, i)),
            pl.BlockSpec((EMB, bb), lambda i: (0, i)),
            pl.BlockSpec((HID, EMB), lambda i: (0, 0)),
            pl.BlockSpec((HID, EMB), lambda i: (0, 0)),
            pl.BlockSpec((HID, 1), lambda i: (0, 0)),
            pl.BlockSpec((1, HID), lambda i: (0, 0)),
            pl.BlockSpec((1, 1), lambda i: (0, 0)),
        ],
        out_specs=pl.BlockSpec((1, bb), lambda i: (0, i)),
        out_shape=jax.ShapeDtypeStruct((1, BATCH), jnp.float32),
    )(ut, vt, w1u, w1v, b1, w2, b2)


def kernel(u, v, user_emb, item_emb, W1, b1, W2, b2):
    u = u.astype(jnp.int32)
    v = v.astype(jnp.int32)
    ue3 = user_emb.reshape(user_emb.shape[0] // 8, 8, EMB)
    ve3 = item_emb.reshape(item_emb.shape[0] // 8, 8, EMB)
    ut, vt = _sc_gather(u, v, ue3, ve3)
    out_t = _tc_mlp(ut, vt, W1[:, :EMB], W1[:, EMB:],
                    b1.reshape(HID, 1), W2, b2.reshape(1, 1))
    return out_t.reshape(BATCH, 1)
